# baseline (speedup 1.0000x reference)
import jax
import jax.numpy as jnp
from jax.experimental import pallas as pl

N, D, H, OUT = 10000, 128, 32, 9
ROWS = 2000


def _dot(a, b):
    return jnp.dot(a, b, preferred_element_type=jnp.float32)


def _sig(v):
    return 0.5 + 0.5 * jnp.tanh(0.5 * v)


def _cell_kernel(x_ref, hc_ref,
                 wxi_ref, whi_ref, wci_ref, bxi_ref, bhi_ref, bi_ref,
                 wxf_ref, whf_ref, wcf_ref, bxf_ref, bhf_ref, bf_ref,
                 wxc_ref, whc_ref, bxc_ref, bhc_ref, bc_ref,
                 wxo_ref, who_ref, wco_ref, bxo_ref, bho_ref, bo_ref,
                 wlin_ref, blin_ref,
                 out_ref):
    x = x_ref[...]
    h = hc_ref[:, 0:H]
    c = hc_ref[:, H:2 * H]
    bi = bxi_ref[...] + bhi_ref[...] + bi_ref[...]
    bf = bxf_ref[...] + bhf_ref[...] + bf_ref[...]
    bc = bxc_ref[...] + bhc_ref[...] + bc_ref[...]
    bo = bxo_ref[...] + bho_ref[...] + bo_ref[...]
    i = jax.nn.sigmoid(_dot(x, wxi_ref[...]) + _dot(h, whi_ref[...])
                       + wci_ref[...] * c + bi)
    f = jax.nn.sigmoid(_dot(x, wxf_ref[...]) + _dot(h, whf_ref[...])
                       + wcf_ref[...] * c + bf)
    t = jnp.tanh(_dot(x, wxc_ref[...]) + _dot(h, whc_ref[...]) + bc)
    c0 = f * c + i * t
    o = jax.nn.sigmoid(_dot(x, wxo_ref[...]) + _dot(h, who_ref[...])
                       + wco_ref[...] * c0 + bo)
    h0 = o * jnp.tanh(c0)
    y = _dot(jax.nn.relu(h0), wlin_ref[...]) + blin_ref[...]
    G1 = jnp.eye(H, D, dtype=jnp.float32)
    G2 = jnp.eye(H, D, k=H, dtype=jnp.float32)
    G3 = jnp.eye(OUT, D, k=2 * H, dtype=jnp.float32)
    out_ref[...] = _dot(h0, G1) + _dot(c0, G2) + _dot(y, G3)


def kernel(x, edge_index, edge_weight, h, c,
           W_xi, b_xi, W_hi, b_hi, w_ci, b_i,
           W_xf, b_xf, W_hf, b_hf, w_cf, b_f,
           W_xc, b_xc, W_hc, b_hc, b_c,
           W_xo, b_xo, W_ho, b_ho, w_co, b_o,
           W_lin, b_lin):
    del edge_index, edge_weight

    E1 = jnp.eye(H, D, dtype=jnp.float32)
    E2 = jnp.eye(H, D, k=H, dtype=jnp.float32)
    hc = _dot(h, E1) + _dot(c, E2)

    grid = (N // ROWS,)
    row_spec = lambda w: pl.BlockSpec((ROWS, w), lambda i: (i, 0))
    full = lambda a: pl.BlockSpec(a.shape, lambda i: (0,) * a.ndim)

    r = lambda b: b.reshape(1, -1)
    ins = (x, hc,
           W_xi, W_hi, w_ci, r(b_xi), r(b_hi), b_i,
           W_xf, W_hf, w_cf, r(b_xf), r(b_hf), b_f,
           W_xc, W_hc, r(b_xc), r(b_hc), b_c,
           W_xo, W_ho, w_co, r(b_xo), r(b_ho), b_o,
           W_lin, r(b_lin))

    packed = pl.pallas_call(
        _cell_kernel,
        grid=grid,
        in_specs=[row_spec(D), row_spec(D)] + [full(a) for a in ins[2:]],
        out_specs=row_spec(D),
        out_shape=jax.ShapeDtypeStruct((N, D), jnp.float32),
    )(*ins)

    F1 = jnp.eye(D, H, dtype=jnp.float32)
    F2 = jnp.eye(D, H, k=-H, dtype=jnp.float32)
    F3 = jnp.eye(D, OUT, k=-2 * H, dtype=jnp.float32)
    h0 = _dot(packed, F1)
    c0 = _dot(packed, F2)
    y = _dot(packed, F3)
    return (y, h0, c0)


# speedup vs baseline: 1.1220x; 1.1220x over previous
import jax
import jax.numpy as jnp
from jax.experimental import pallas as pl

N, D, H, OUT = 10000, 128, 32, 9
ROWS = 2000


def _dot(a, b):
    return jnp.dot(a, b, preferred_element_type=jnp.float32)


def _cell_kernel(x_ref, hc_ref,
                 wxi_ref, whi_ref, wci_ref, bxi_ref, bhi_ref, bi_ref,
                 wxf_ref, whf_ref, wcf_ref, bxf_ref, bhf_ref, bf_ref,
                 wxc_ref, whc_ref, bxc_ref, bhc_ref, bc_ref,
                 wxo_ref, who_ref, wco_ref, bxo_ref, bho_ref, bo_ref,
                 wlin_ref, blin_ref,
                 out_ref):
    x = x_ref[...]
    h = hc_ref[:, 0:H]
    c = hc_ref[:, H:2 * H]
    bi = bxi_ref[...] + bhi_ref[...] + bi_ref[...]
    bf = bxf_ref[...] + bhf_ref[...] + bf_ref[...]
    bc = bxc_ref[...] + bhc_ref[...] + bc_ref[...]
    bo = bxo_ref[...] + bho_ref[...] + bo_ref[...]
    i = jax.nn.sigmoid(_dot(x, wxi_ref[...]) + _dot(h, whi_ref[...])
                       + wci_ref[...] * c + bi)
    f = jax.nn.sigmoid(_dot(x, wxf_ref[...]) + _dot(h, whf_ref[...])
                       + wcf_ref[...] * c + bf)
    t = jnp.tanh(_dot(x, wxc_ref[...]) + _dot(h, whc_ref[...]) + bc)
    c0 = f * c + i * t
    o = jax.nn.sigmoid(_dot(x, wxo_ref[...]) + _dot(h, who_ref[...])
                       + wco_ref[...] * c0 + bo)
    h0 = o * jnp.tanh(c0)
    y = _dot(jax.nn.relu(h0), wlin_ref[...]) + blin_ref[...]
    z = jnp.zeros((x.shape[0], 2 * H - OUT), jnp.float32)
    out_ref[...] = jnp.concatenate([h0, c0, y, z], axis=1)


def kernel(x, edge_index, edge_weight, h, c,
           W_xi, b_xi, W_hi, b_hi, w_ci, b_i,
           W_xf, b_xf, W_hf, b_hf, w_cf, b_f,
           W_xc, b_xc, W_hc, b_hc, b_c,
           W_xo, b_xo, W_ho, b_ho, w_co, b_o,
           W_lin, b_lin):
    del edge_index, edge_weight

    E1 = jnp.eye(H, D, dtype=jnp.float32)
    E2 = jnp.eye(H, D, k=H, dtype=jnp.float32)
    hc = _dot(h, E1) + _dot(c, E2)

    grid = (N // ROWS,)
    row_spec = lambda w: pl.BlockSpec((ROWS, w), lambda i: (i, 0))
    full = lambda a: pl.BlockSpec(a.shape, lambda i: (0,) * a.ndim)

    r = lambda b: b.reshape(1, -1)
    ins = (x, hc,
           W_xi, W_hi, w_ci, r(b_xi), r(b_hi), b_i,
           W_xf, W_hf, w_cf, r(b_xf), r(b_hf), b_f,
           W_xc, W_hc, r(b_xc), r(b_hc), b_c,
           W_xo, W_ho, w_co, r(b_xo), r(b_ho), b_o,
           W_lin, r(b_lin))

    packed = pl.pallas_call(
        _cell_kernel,
        grid=grid,
        in_specs=[row_spec(D), row_spec(D)] + [full(a) for a in ins[2:]],
        out_specs=row_spec(D),
        out_shape=jax.ShapeDtypeStruct((N, D), jnp.float32),
    )(*ins)

    F1 = jnp.eye(D, H, dtype=jnp.float32)
    F2 = jnp.eye(D, H, k=-H, dtype=jnp.float32)
    F3 = jnp.eye(D, OUT, k=-2 * H, dtype=jnp.float32)
    h0 = _dot(packed, F1)
    c0 = _dot(packed, F2)
    y = _dot(packed, F3)
    return (y, h0, c0)
